# baseline (speedup 1.0000x reference)
import dataclasses

import jax
import jax.numpy as jnp
from jax import lax
from jax.experimental import pallas as pl
from jax.experimental.pallas import tpu as pltpu
from jax.experimental.pallas import tpu_sc as plsc

_N = 524288
_R = 4096
_B = 16384
_NBLK = _N // _B
_NCH = 32
_CS = _N // _NCH
_SC_STEP = 16384
_SC_NSTEP = _CS // _SC_STEP


def _tc1_body(feat, ts, dtl, ray, W1T, b1c, WoutT, boutT,
              st8_o, mm_o, smf, smi):
    pid = pl.program_id(0)

    @pl.when(pid == 0)
    def _():
        smf[0] = jnp.float32(0.0)
        smf[1] = jnp.float32(jnp.inf)
        smf[2] = jnp.float32(-jnp.inf)
        smf[3] = jnp.float32(0.0)
        smi[0] = jnp.int32(-1)

    xT = jnp.swapaxes(feat[...], 0, 1)
    hT = jnp.maximum(
        jnp.dot(W1T[...], xT, preferred_element_type=jnp.float32)
        + b1c[...], 0.0)
    oT = jnp.dot(WoutT[...], hT,
                 preferred_element_type=jnp.float32) + boutT[...]
    rr = jax.nn.sigmoid(oT[0:1])
    gg = jax.nn.sigmoid(oT[1:2])
    bb = jax.nn.sigmoid(oT[2:3])
    sigma = jnp.exp(jnp.clip(oT[3:4], -15.0, 15.0))

    dt2 = dtl[...].reshape(1, _B)
    ts2 = ts[...].reshape(1, _B)
    s2 = sigma * dt2
    al2 = 1.0 - jnp.exp(-s2)
    st2 = ts2 + 0.5 * dt2

    incl = s2
    d = 1
    while d < _B:
        incl = incl + jnp.concatenate(
            [jnp.zeros((1, d), jnp.float32), incl[:, :_B - d]], axis=1)
        d *= 2
    carry = smf[0]
    ecs = incl - s2 + carry
    smf[0] = carry + jnp.max(incl)

    ray2 = ray[...].reshape(1, _B)
    top = jnp.full((1, 1), smi[0], jnp.int32)
    prev = jnp.concatenate([top, ray2[:, :_B - 1]], axis=1)
    m2 = jnp.where(ray2 != prev, ecs, -1.0)
    smi[0] = jnp.max(ray2)

    ff = m2
    d = 1
    while d < _B:
        ff = jnp.maximum(ff, jnp.concatenate(
            [jnp.full((1, d), -1.0, jnp.float32), ff[:, :_B - d]], axis=1))
        d *= 2
    off = jnp.maximum(ff, smf[3])
    smf[3] = jnp.max(off)

    w = jnp.exp(off - ecs) * al2
    wr = w * rr
    wg = w * gg
    wb = w * bb
    wst = w * st2

    newmin = jnp.minimum(smf[1], jnp.min(st2))
    newmax = jnp.maximum(smf[2], jnp.max(st2))
    smf[1] = newmin
    smf[2] = newmax
    lane1 = lax.broadcasted_iota(jnp.int32, (1, 128), 1)
    mm_o[...] = jnp.where(lane1 == 0, newmin, newmax)

    st8_o[...] = jnp.concatenate(
        [wr, wg, wb, w, wst, w, w, w], axis=0).reshape(
            8, _B // 2048, 2048)


def _tc1(feat, ts3, dt3, ray3, W1T, b1c, WoutT, boutT):
    f32 = jnp.float32
    vspec = pl.BlockSpec((1, 1, _B), lambda i: (i, 0, 0))
    full = lambda shp: pl.BlockSpec(shp, lambda i: (0,) * len(shp))
    return pl.pallas_call(
        _tc1_body,
        grid=(_NBLK,),
        in_specs=[
            pl.BlockSpec((_B, 64), lambda i: (i, 0)),
            vspec, vspec, vspec,
            full((128, 64)), full((128, 1)), full((8, 128)), full((8, 1)),
        ],
        out_specs=[pl.BlockSpec((8, _B // 2048, 2048), lambda i: (0, i, 0)),
                   full((1, 128))],
        out_shape=[jax.ShapeDtypeStruct((8, _N // 2048, 2048), f32),
                   jax.ShapeDtypeStruct((1, 128), f32)],
        scratch_shapes=[pltpu.SMEM((4,), jnp.float32),
                        pltpu.SMEM((1,), jnp.int32)],
    )(feat, ts3, dt3, ray3, W1T, b1c, WoutT, boutT)


def _sc_body(st8_h, ray_h,
             pr_o, pg_o, pb_o, pw_o, ps_o,
             rbuf, gbuf, bbuf, wbuf, sbuf, ybuf,
             tr, tg, tb, tw, td, sems):
    c = lax.axis_index("c")
    s = lax.axis_index("s")
    tid = c * 16 + s
    base = tid * _CS

    z16 = jnp.zeros((16,), jnp.float32)

    @pl.loop(0, _R // 16)
    def _(j):
        sl = pl.ds(j * 16, 16)
        tr[sl] = z16
        tg[sl] = z16
        tb[sl] = z16
        tw[sl] = z16
        td[sl] = z16

    for step in range(_SC_NSTEP):
        off = base + step * _SC_STEP
        bufs = [rbuf, gbuf, bbuf, wbuf, sbuf]
        cps = [pltpu.async_copy(st8_h.at[k, pl.ds(off, _SC_STEP)], buf,
                                sems.at[k])
               for k, buf in enumerate(bufs)]
        cps.append(pltpu.async_copy(ray_h.at[pl.ds(off, _SC_STEP)], ybuf,
                                    sems.at[5]))
        for cp in cps:
            cp.wait()

        @pl.loop(0, _SC_STEP // 16)
        def _(j):
            sl = pl.ds(j * 16, 16)
            y16 = ybuf[sl]
            plsc.addupdate_scatter(tr, [y16], rbuf[sl])
            plsc.addupdate_scatter(tg, [y16], gbuf[sl])
            plsc.addupdate_scatter(tb, [y16], bbuf[sl])
            plsc.addupdate_scatter(tw, [y16], wbuf[sl])
            plsc.addupdate_scatter(td, [y16], sbuf[sl])

    pltpu.sync_copy(tr, pr_o.at[tid])
    pltpu.sync_copy(tg, pg_o.at[tid])
    pltpu.sync_copy(tb, pb_o.at[tid])
    pltpu.sync_copy(tw, pw_o.at[tid])
    pltpu.sync_copy(td, ps_o.at[tid])


def _sc(st8, ray):
    f32 = jnp.float32
    part = jax.ShapeDtypeStruct((_NCH, _R), f32)
    vbuf = pltpu.VMEM((_SC_STEP,), f32)
    cp = pltpu.CompilerParams()
    if "needs_layout_passes" in pltpu.CompilerParams.__dataclass_fields__:
        cp = dataclasses.replace(cp, needs_layout_passes=False)
    ker = pl.kernel(
        _sc_body,
        out_type=[part] * 5,
        compiler_params=cp,
        mesh=plsc.VectorSubcoreMesh(core_axis_name="c", subcore_axis_name="s"),
        scratch_types=[vbuf, vbuf, vbuf, vbuf, vbuf,
                       pltpu.VMEM((_SC_STEP,), jnp.int32)]
                      + [pltpu.VMEM((_R,), f32)] * 5
                      + [pltpu.SemaphoreType.DMA((6,))],
    )
    return ker(st8, ray)


def _fin_body(pr, pg, pb, pw, ps, bgT, mm, rgb_o, dep_o, acc_o):
    def tot(p):
        return jnp.sum(p[...], axis=0, keepdims=True)

    Wt = tot(pw)
    Rt = tot(pr)
    Gt = tot(pg)
    Bt = tot(pb)
    St = tot(ps)
    one_m = 1.0 - Wt
    bg = bgT[...]
    cr = Rt + one_m * bg[0:1]
    cg = Gt + one_m * bg[1:2]
    cb = Bt + one_m * bg[2:3]
    rgb_o[...] = jnp.swapaxes(jnp.concatenate([cr, cg, cb], axis=0), 0, 1)
    mn = mm[0:1, 0:1]
    mx = mm[0:1, 1:2]
    d = St / (Wt + 1e-10)
    dep_o[...] = jnp.swapaxes(jnp.minimum(jnp.maximum(d, mn), mx), 0, 1)
    acc_o[...] = jnp.swapaxes(Wt, 0, 1)


def _fin(pr, pg, pb, pw, ps, bgT, mm):
    f32 = jnp.float32
    return pl.pallas_call(
        _fin_body,
        out_shape=[jax.ShapeDtypeStruct((_R, 3), f32),
                   jax.ShapeDtypeStruct((_R, 1), f32),
                   jax.ShapeDtypeStruct((_R, 1), f32)],
    )(pr, pg, pb, pw, ps, bgT, mm)


def kernel(features, t_starts, t_deltas, ray_indices, bg_color,
           W1, b1, W_rgb, b_rgb, W_sigma, b_sigma):
    f32 = jnp.float32
    ts3 = t_starts.reshape(_NBLK, 1, _B)
    dt3 = t_deltas.reshape(_NBLK, 1, _B)
    ray3 = ray_indices.reshape(_NBLK, 1, _B)
    WoutT = jnp.concatenate(
        [W_rgb, W_sigma, jnp.zeros((128, 4), f32)], axis=1).T
    boutT = jnp.concatenate([b_rgb, b_sigma, jnp.zeros((4,), f32)]).reshape(8, 1)
    W1T = W1.T
    b1c = b1.reshape(128, 1)

    st8_3, mm = _tc1(features, ts3, dt3, ray3, W1T, b1c, WoutT, boutT)
    st8 = st8_3.reshape(8, _N)

    pr, pg, pb, pw, ps = _sc(st8, ray_indices)

    bgT = bg_color.T
    rgb, dep, acc = _fin(pr, pg, pb, pw, ps, bgT, mm)
    return rgb, dep, acc


# speedup vs baseline: 1.0435x; 1.0435x over previous
import dataclasses

import jax
import jax.numpy as jnp
from jax import lax
from jax.experimental import pallas as pl
from jax.experimental.pallas import tpu as pltpu
from jax.experimental.pallas import tpu_sc as plsc

_N = 524288
_R = 4096
_B = 16384
_NBLK = _N // _B
_NCH = 32
_CS = _N // _NCH
_SC_STEP = 8192
_SC_NSTEP = _CS // _SC_STEP


def _tc1_body(feat, ts, dtl, ray, W1T, b1c, WoutT, boutT,
              st8_o, mm_o, smf, smi):
    pid = pl.program_id(0)

    @pl.when(pid == 0)
    def _():
        smf[0] = jnp.float32(0.0)
        smf[1] = jnp.float32(jnp.inf)
        smf[2] = jnp.float32(-jnp.inf)
        smi[0] = jnp.int32(-1)

    xT = jnp.swapaxes(feat[...], 0, 1)
    hT = jnp.maximum(
        jnp.dot(W1T[...], xT, preferred_element_type=jnp.float32)
        + b1c[...], 0.0)
    oT = jnp.dot(WoutT[...], hT,
                 preferred_element_type=jnp.float32) + boutT[...]
    rr = jax.nn.sigmoid(oT[0:1])
    gg = jax.nn.sigmoid(oT[1:2])
    bb = jax.nn.sigmoid(oT[2:3])
    sigma = jnp.exp(jnp.clip(oT[3:4], -15.0, 15.0))

    dt2 = dtl[...].reshape(1, _B)
    ts2 = ts[...].reshape(1, _B)
    s2 = sigma * dt2
    al2 = 1.0 - jnp.exp(-s2)
    st2 = ts2 + 0.5 * dt2

    incl = s2
    d = 1
    while d < _B:
        incl = incl + jnp.concatenate(
            [jnp.zeros((1, d), jnp.float32), incl[:, :_B - d]], axis=1)
        d *= 2
    carry = smf[0]
    ecs = incl - s2 + carry
    smf[0] = carry + jnp.max(incl)

    ray2 = ray[...].reshape(1, _B)
    top = jnp.full((1, 1), smi[0], jnp.int32)
    prev = jnp.concatenate([top, ray2[:, :_B - 1]], axis=1)
    m2 = jnp.where(ray2 != prev, ecs, -1.0)
    smi[0] = jnp.max(ray2)

    newmin = jnp.minimum(smf[1], jnp.min(st2))
    newmax = jnp.maximum(smf[2], jnp.max(st2))
    smf[1] = newmin
    smf[2] = newmax
    lane1 = lax.broadcasted_iota(jnp.int32, (1, 128), 1)
    mm_o[...] = jnp.where(lane1 == 0, newmin, newmax)

    st8_o[...] = jnp.concatenate(
        [ecs, m2, al2, rr, gg, bb, st2, st2], axis=0).reshape(
            8, _B // 2048, 2048)


def _tc1(feat, ts3, dt3, ray3, W1T, b1c, WoutT, boutT):
    f32 = jnp.float32
    vspec = pl.BlockSpec((1, 1, _B), lambda i: (i, 0, 0))
    full = lambda shp: pl.BlockSpec(shp, lambda i: (0,) * len(shp))
    return pl.pallas_call(
        _tc1_body,
        grid=(_NBLK,),
        in_specs=[
            pl.BlockSpec((_B, 64), lambda i: (i, 0)),
            vspec, vspec, vspec,
            full((128, 64)), full((128, 1)), full((8, 128)), full((8, 1)),
        ],
        out_specs=[pl.BlockSpec((8, _B // 2048, 2048), lambda i: (0, i, 0)),
                   full((1, 128))],
        out_shape=[jax.ShapeDtypeStruct((8, _N // 2048, 2048), f32),
                   jax.ShapeDtypeStruct((1, 128), f32)],
        scratch_shapes=[pltpu.SMEM((3,), jnp.float32),
                        pltpu.SMEM((1,), jnp.int32)],
    )(feat, ts3, dt3, ray3, W1T, b1c, WoutT, boutT)


def _sc_body(st8_h, ray_h,
             pr_o, pg_o, pb_o, pw_o, ps_o,
             ebuf, mbuf, abuf, rbuf, gbuf, bbuf, sbuf, ybuf,
             tr, tg, tb, tw, td, car, sems):
    c = lax.axis_index("c")
    s = lax.axis_index("s")
    tid = c * 16 + s
    base = tid * _CS

    z16 = jnp.zeros((16,), jnp.float32)

    @pl.loop(0, _R // 16)
    def _(j):
        sl = pl.ds(j * 16, 16)
        tr[sl] = z16
        tg[sl] = z16
        tb[sl] = z16
        tw[sl] = z16
        td[sl] = z16

    for step in range(_SC_NSTEP):
        off = base + step * _SC_STEP
        bufs = [ebuf, mbuf, abuf, rbuf, gbuf, bbuf, sbuf]
        cps = [pltpu.async_copy(st8_h.at[k, pl.ds(off, _SC_STEP)], buf,
                                sems.at[k])
               for k, buf in enumerate(bufs)]
        cps.append(pltpu.async_copy(ray_h.at[pl.ds(off, _SC_STEP)], ybuf,
                                    sems.at[7]))
        for cp in cps:
            cp.wait()
        if step == 0:
            e0 = ebuf[pl.ds(0, 16)]
            car[0] = e0[0]

        @pl.loop(0, _SC_STEP // 16)
        def _(j):
            sl = pl.ds(j * 16, 16)
            e16 = ebuf[sl]
            cm = plsc.cummax(mbuf[sl])
            offv = jnp.maximum(cm, car[0])
            car[0] = jnp.max(offv)
            w = jnp.exp(offv - e16) * abuf[sl]
            y16 = ybuf[sl]
            plsc.addupdate_scatter(tw, [y16], w)
            plsc.addupdate_scatter(tr, [y16], w * rbuf[sl])
            plsc.addupdate_scatter(tg, [y16], w * gbuf[sl])
            plsc.addupdate_scatter(tb, [y16], w * bbuf[sl])
            plsc.addupdate_scatter(td, [y16], w * sbuf[sl])

    pltpu.sync_copy(tr, pr_o.at[tid])
    pltpu.sync_copy(tg, pg_o.at[tid])
    pltpu.sync_copy(tb, pb_o.at[tid])
    pltpu.sync_copy(tw, pw_o.at[tid])
    pltpu.sync_copy(td, ps_o.at[tid])


def _sc(st8, ray):
    f32 = jnp.float32
    part = jax.ShapeDtypeStruct((_NCH, _R), f32)
    vbuf = pltpu.VMEM((_SC_STEP,), f32)
    cp = pltpu.CompilerParams()
    if "needs_layout_passes" in pltpu.CompilerParams.__dataclass_fields__:
        cp = dataclasses.replace(cp, needs_layout_passes=False)
    ker = pl.kernel(
        _sc_body,
        out_type=[part] * 5,
        compiler_params=cp,
        mesh=plsc.VectorSubcoreMesh(core_axis_name="c", subcore_axis_name="s"),
        scratch_types=[vbuf, vbuf, vbuf, vbuf, vbuf, vbuf, vbuf,
                       pltpu.VMEM((_SC_STEP,), jnp.int32)]
                      + [pltpu.VMEM((_R,), f32)] * 5
                      + [pltpu.SMEM((1,), f32),
                         pltpu.SemaphoreType.DMA((8,))],
    )
    return ker(st8, ray)


def _fin_body(pr, pg, pb, pw, ps, m_f, ecs_f, hr, bgT, mm,
              rgb_o, dep_o, acc_o):
    m3 = m_f[...].reshape(_NCH, 128, 128)
    ecs3 = ecs_f[...].reshape(_NCH, 128, 128)
    chmax = jnp.max(jnp.max(m3, axis=2), axis=1, keepdims=True)

    lane = lax.broadcasted_iota(jnp.int32, (_NCH, 1, 128), 2)

    def first_lane(x3):
        row0 = x3[:, 0:1, :]
        return jnp.sum(jnp.where(lane == 0, row0, 0.0), axis=2)

    ecs_start = first_lane(ecs3)
    m_start = first_lane(m3)

    ex = jnp.concatenate([jnp.zeros((1, 1), jnp.float32), chmax[:_NCH - 1]],
                         axis=0)
    for d in (1, 2, 4, 8, 16):
        ex = jnp.maximum(
            ex, jnp.concatenate([jnp.full((d, 1), -1.0, jnp.float32),
                                 ex[:_NCH - d]], axis=0))
    corr = jnp.where(m_start >= 0.0, 1.0, jnp.exp(-(ecs_start - ex)))

    hr2 = hr[...]
    colj = lax.broadcasted_iota(jnp.int32, (_NCH, _R), 1)
    scale = jnp.where(colj == hr2, corr, 1.0)

    def tot(p):
        return jnp.sum(p[...] * scale, axis=0, keepdims=True)

    Wt = tot(pw)
    Rt = tot(pr)
    Gt = tot(pg)
    Bt = tot(pb)
    St = tot(ps)
    one_m = 1.0 - Wt
    bg = bgT[...]
    cr = Rt + one_m * bg[0:1]
    cg = Gt + one_m * bg[1:2]
    cb = Bt + one_m * bg[2:3]
    rgb_o[...] = jnp.concatenate([cr, cg, cb], axis=0)
    mn = mm[0:1, 0:1]
    mx = mm[0:1, 1:2]
    d = St / (Wt + 1e-10)
    dep_o[...] = jnp.minimum(jnp.maximum(d, mn), mx)
    acc_o[...] = Wt


def _fin(pr, pg, pb, pw, ps, m2, ecs2, hr, bgT, mm):
    f32 = jnp.float32
    return pl.pallas_call(
        _fin_body,
        out_shape=[jax.ShapeDtypeStruct((3, _R), f32),
                   jax.ShapeDtypeStruct((1, _R), f32),
                   jax.ShapeDtypeStruct((1, _R), f32)],
    )(pr, pg, pb, pw, ps, m2, ecs2, hr, bgT, mm)


def kernel(features, t_starts, t_deltas, ray_indices, bg_color,
           W1, b1, W_rgb, b_rgb, W_sigma, b_sigma):
    f32 = jnp.float32
    ts3 = t_starts.reshape(_NBLK, 1, _B)
    dt3 = t_deltas.reshape(_NBLK, 1, _B)
    ray3 = ray_indices.reshape(_NBLK, 1, _B)
    WoutT = jnp.concatenate(
        [W_rgb, W_sigma, jnp.zeros((128, 4), f32)], axis=1).T
    boutT = jnp.concatenate([b_rgb, b_sigma, jnp.zeros((4,), f32)]).reshape(8, 1)
    W1T = W1.T
    b1c = b1.reshape(128, 1)

    st8_3, mm = _tc1(features, ts3, dt3, ray3, W1T, b1c, WoutT, boutT)
    st8 = st8_3.reshape(8, _N)

    pr, pg, pb, pw, ps = _sc(st8, ray_indices)

    hr = ray_indices[::_CS].reshape(_NCH, 1)
    bgT = bg_color.T
    m2 = st8[1].reshape(_N // 128, 128)
    ecs2 = st8[0].reshape(_N // 128, 128)
    rgbT, depT, accT = _fin(pr, pg, pb, pw, ps, m2, ecs2, hr, bgT, mm)
    return rgbT.T, depT.T, accT.T
